# baseline (speedup 1.0000x reference)
import functools

import jax
import jax.numpy as jnp
from jax import lax
from jax.experimental import pallas as pl
from jax.experimental.pallas import tpu as pltpu
from jax.experimental.pallas import tpu_sc as plsc

NC = 2
NS = 16
NW = NC * NS

CHUNK = 128


def _tc_project(x, wt, bt, wp, bp):
    N, D = x.shape
    BL = 1000
    grid = (N // BL,)

    def body(x_ref, wt_ref, bt_ref, wp_ref, bp_ref, z0_ref, zc_ref):
        m = jnp.dot(wt_ref[...], wp_ref[...], preferred_element_type=jnp.float32)
        c = jnp.dot(bt_ref[...], wp_ref[...], preferred_element_type=jnp.float32)
        c = c + bp_ref[...]
        z0 = jnp.dot(x_ref[...], m, preferred_element_type=jnp.float32)
        z0_ref[...] = z0
        zc_ref[...] = z0 + c

    row = pl.BlockSpec((BL, D), lambda i: (i, 0))
    full = pl.BlockSpec((D, D), lambda i: (0, 0))
    vec = pl.BlockSpec((1, D), lambda i: (0, 0))
    return pl.pallas_call(
        body,
        grid=grid,
        in_specs=[row, full, vec, full, vec],
        out_specs=[row, row],
        out_shape=[
            jax.ShapeDtypeStruct((N, D), jnp.float32),
            jax.ShapeDtypeStruct((N, D), jnp.float32),
        ],
    )(x, wt, bt, wp, bp)


def _sc_finish(z0, zc, src, dst, iota, V):
    N, D = z0.shape
    E = src.shape[0]
    ew = E // NS
    nch = ew // CHUNK
    slab = 624
    tail = N - NS * slab

    mesh = plsc.VectorSubcoreMesh(
        core_axis_name="c", subcore_axis_name="s", num_cores=1, num_subcores=NS
    )

    OCH = 104

    @functools.partial(
        pl.kernel,
        mesh=mesh,
        out_type=jax.ShapeDtypeStruct((N, D), jnp.float32),
        scratch_types=[
            pltpu.VMEM_SHARED((N + 8, D), jnp.float32),
            pltpu.VMEM((CHUNK,), jnp.int32),
            pltpu.VMEM((CHUNK,), jnp.int32),
            pltpu.VMEM((CHUNK,), jnp.int32),
            pltpu.VMEM((CHUNK,), jnp.int32),
            pltpu.VMEM((CHUNK, D), jnp.float32),
            pltpu.VMEM((CHUNK, D), jnp.float32),
            pltpu.SemaphoreType.DMA,
            pltpu.SemaphoreType.DMA,
            pltpu.SemaphoreType.DMA,
            pltpu.SemaphoreType.DMA,
        ],
    )
    def k(z0_hbm, zc_hbm, src_hbm, dst_hbm, iota_hbm, out_hbm, agg_sh,
          src_v0, dst_v0, src_v1, dst_v1, rows0, rows1,
          sem_i0, sem_i1, sem_g0, sem_g1):
        sid = lax.axis_index("s")
        wid = sid

        r0 = sid * slab
        pltpu.sync_copy(zc_hbm.at[pl.ds(r0, slab)], agg_sh.at[pl.ds(r0, slab)])

        @pl.when(sid == NS - 1)
        def _():
            pltpu.sync_copy(
                zc_hbm.at[pl.ds(NS * slab, tail)], agg_sh.at[pl.ds(NS * slab, tail)]
            )

        plsc.subcore_barrier()

        ebase = wid * ew

        def fire_idx(c, sv, dv, sem):
            base = ebase + c * CHUNK
            pltpu.async_copy(src_hbm.at[pl.ds(base, CHUNK)], sv, sem)
            pltpu.async_copy(dst_hbm.at[pl.ds(base, CHUNK)], dv, sem)

        def wait_idx(sv, dv, sem):
            pltpu.make_async_copy(src_hbm.at[pl.ds(0, CHUNK)], sv, sem).wait()
            pltpu.make_async_copy(dst_hbm.at[pl.ds(0, CHUNK)], dv, sem).wait()

        fire_idx(0, src_v0, dst_v0, sem_i0)
        fire_idx(1, src_v1, dst_v1, sem_i1)
        wait_idx(src_v0, dst_v0, sem_i0)
        pltpu.async_copy(z0_hbm.at[src_v0], rows0, sem_g0)

        @pl.loop(0, nch // 2)
        def _(i):
            more = i < nch // 2 - 1
            wait_idx(src_v1, dst_v1, sem_i1)
            pltpu.async_copy(z0_hbm.at[src_v1], rows1, sem_g1)
            pltpu.make_async_copy(z0_hbm.at[src_v0], rows0, sem_g0).wait()
            pltpu.sync_copy(rows0, agg_sh.at[dst_v0], add=True)

            @pl.when(more)
            def _():
                fire_idx(2 * i + 2, src_v0, dst_v0, sem_i0)
                wait_idx(src_v0, dst_v0, sem_i0)
                pltpu.async_copy(z0_hbm.at[src_v0], rows0, sem_g0)

            pltpu.make_async_copy(z0_hbm.at[src_v1], rows1, sem_g1).wait()
            pltpu.sync_copy(rows1, agg_sh.at[dst_v1], add=True)

            @pl.when(more)
            def _():
                fire_idx(2 * i + 3, src_v1, dst_v1, sem_i1)

        plsc.subcore_barrier()

        pltpu.sync_copy(agg_sh.at[pl.ds(r0, slab)], out_hbm.at[pl.ds(r0, slab)])

        @pl.when(sid == NS - 1)
        def _():
            t0 = NS * slab
            pltpu.sync_copy(agg_sh.at[pl.ds(t0, tail)], out_hbm.at[pl.ds(t0, tail)])

    return k(z0, zc, src, dst, iota)


def kernel(x, edge_index, W_trn, b_trn, W_prd, b_prd):
    N, D = x.shape
    V = W_prd.shape[0]

    wt = W_trn.T
    wp = jnp.pad(W_prd, ((0, D - V), (0, 0))).T
    bt = b_trn.reshape(1, D)
    bp = jnp.pad(b_prd, (0, D - V)).reshape(1, D)
    z0, zc = _tc_project(x, wt, bt, wp, bp)

    E = edge_index.shape[1]
    per_tile = -(-E // (NS * 2 * CHUNK)) * 2 * CHUNK
    EP = NS * per_tile
    src = jnp.pad(edge_index[0], (0, EP - E))
    dst = jnp.pad(edge_index[1], (0, EP - E), constant_values=N)

    iota = jnp.arange(N, dtype=jnp.int32)
    out_full = _sc_finish(z0, zc, src, dst, iota, V)
    return out_full[:, :V]


# speedup vs baseline: 1.2999x; 1.2999x over previous
import functools

import jax
import jax.numpy as jnp
from jax import lax
from jax.experimental import pallas as pl
from jax.experimental.pallas import tpu as pltpu
from jax.experimental.pallas import tpu_sc as plsc

NC = 2
NS = 16
NW = NC * NS

CHUNK = 80


def _tc_project(x, wt, bt, wp, bp):
    N, D = x.shape
    BL = 1000
    grid = (N // BL,)

    def body(x_ref, wt_ref, bt_ref, wp_ref, bp_ref, z0_ref, zc_ref):
        m = jnp.dot(wt_ref[...], wp_ref[...], preferred_element_type=jnp.float32)
        c = jnp.dot(bt_ref[...], wp_ref[...], preferred_element_type=jnp.float32)
        c = c + bp_ref[...]
        z0 = jnp.dot(x_ref[...], m, preferred_element_type=jnp.float32)
        z0_ref[...] = z0
        zc_ref[...] = z0 + c

    row = pl.BlockSpec((BL, D), lambda i: (i, 0))
    full = pl.BlockSpec((D, D), lambda i: (0, 0))
    vec = pl.BlockSpec((1, D), lambda i: (0, 0))
    return pl.pallas_call(
        body,
        grid=grid,
        in_specs=[row, full, vec, full, vec],
        out_specs=[row, row],
        out_shape=[
            jax.ShapeDtypeStruct((N, D), jnp.float32),
            jax.ShapeDtypeStruct((N, D), jnp.float32),
        ],
    )(x, wt, bt, wp, bp)


def _sc_finish(z0, zc, src, dst, iota, V):
    N, D = z0.shape
    E = src.shape[0]
    ew = E // NS
    nch = ew // CHUNK
    slab = 624
    tail = N - NS * slab

    mesh = plsc.VectorSubcoreMesh(
        core_axis_name="c", subcore_axis_name="s", num_cores=1, num_subcores=NS
    )

    OCH = 104

    @functools.partial(
        pl.kernel,
        mesh=mesh,
        out_type=jax.ShapeDtypeStruct((N, D), jnp.float32),
        scratch_types=[
            pltpu.VMEM_SHARED((N + 8, D), jnp.float32),
            pltpu.VMEM((CHUNK,), jnp.int32),
            pltpu.VMEM((CHUNK,), jnp.int32),
            pltpu.VMEM((CHUNK,), jnp.int32),
            pltpu.VMEM((CHUNK,), jnp.int32),
            pltpu.VMEM((CHUNK, D), jnp.float32),
            pltpu.VMEM((CHUNK, D), jnp.float32),
            pltpu.SemaphoreType.DMA,
            pltpu.SemaphoreType.DMA,
            pltpu.SemaphoreType.DMA,
            pltpu.SemaphoreType.DMA,
        ],
    )
    def k(z0_hbm, zc_hbm, src_hbm, dst_hbm, iota_hbm, out_hbm, agg_sh,
          src_v0, dst_v0, src_v1, dst_v1, rows0, rows1,
          sem_i0, sem_i1, sem_g0, sem_g1):
        sid = lax.axis_index("s")
        wid = sid

        r0 = sid * slab
        pltpu.sync_copy(zc_hbm.at[pl.ds(r0, slab)], agg_sh.at[pl.ds(r0, slab)])

        @pl.when(sid == NS - 1)
        def _():
            pltpu.sync_copy(
                zc_hbm.at[pl.ds(NS * slab, tail)], agg_sh.at[pl.ds(NS * slab, tail)]
            )

        plsc.subcore_barrier()

        ebase = wid * ew

        def fire_idx(c, sv, dv, sem):
            base = ebase + c * CHUNK
            pltpu.async_copy(src_hbm.at[pl.ds(base, CHUNK)], sv, sem)
            pltpu.async_copy(dst_hbm.at[pl.ds(base, CHUNK)], dv, sem)

        def wait_idx(sv, dv, sem):
            pltpu.make_async_copy(src_hbm.at[pl.ds(0, CHUNK)], sv, sem).wait()
            pltpu.make_async_copy(dst_hbm.at[pl.ds(0, CHUNK)], dv, sem).wait()

        fire_idx(0, src_v0, dst_v0, sem_i0)
        fire_idx(1, src_v1, dst_v1, sem_i1)
        wait_idx(src_v0, dst_v0, sem_i0)
        pltpu.async_copy(z0_hbm.at[src_v0], rows0, sem_g0)

        @pl.loop(0, nch // 2)
        def _(i):
            more = i < nch // 2 - 1
            wait_idx(src_v1, dst_v1, sem_i1)
            pltpu.async_copy(z0_hbm.at[src_v1], rows1, sem_g1)
            pltpu.make_async_copy(z0_hbm.at[src_v0], rows0, sem_g0).wait()
            pltpu.sync_copy(rows0, agg_sh.at[dst_v0], add=True)

            @pl.when(more)
            def _():
                fire_idx(2 * i + 2, src_v0, dst_v0, sem_i0)
                wait_idx(src_v0, dst_v0, sem_i0)
                pltpu.async_copy(z0_hbm.at[src_v0], rows0, sem_g0)

            pltpu.make_async_copy(z0_hbm.at[src_v1], rows1, sem_g1).wait()
            pltpu.sync_copy(rows1, agg_sh.at[dst_v1], add=True)

            @pl.when(more)
            def _():
                fire_idx(2 * i + 3, src_v1, dst_v1, sem_i1)

        plsc.subcore_barrier()

        pltpu.sync_copy(agg_sh.at[pl.ds(r0, slab)], out_hbm.at[pl.ds(r0, slab)])

        @pl.when(sid == NS - 1)
        def _():
            t0 = NS * slab
            pltpu.sync_copy(agg_sh.at[pl.ds(t0, tail)], out_hbm.at[pl.ds(t0, tail)])

    return k(z0, zc, src, dst, iota)


def kernel(x, edge_index, W_trn, b_trn, W_prd, b_prd):
    N, D = x.shape
    V = W_prd.shape[0]

    wt = W_trn.T
    wp = jnp.pad(W_prd, ((0, D - V), (0, 0))).T
    bt = b_trn.reshape(1, D)
    bp = jnp.pad(b_prd, (0, D - V)).reshape(1, D)
    z0, zc = _tc_project(x, wt, bt, wp, bp)

    E = edge_index.shape[1]
    per_tile = -(-E // (NS * 2 * CHUNK)) * 2 * CHUNK
    EP = NS * per_tile
    src = jnp.pad(edge_index[0], (0, EP - E))
    dst = jnp.pad(edge_index[1], (0, EP - E), constant_values=N)

    iota = jnp.arange(N, dtype=jnp.int32)
    out_full = _sc_finish(z0, zc, src, dst, iota, V)
    return out_full[:, :V]
